# baseline (speedup 1.0000x reference)
import jax
import jax.numpy as jnp
from jax import lax
from jax.experimental import pallas as pl
from jax.experimental.pallas import tpu as pltpu
from jax.experimental.pallas import tpu_sc as plsc

VOCAB = 100000
EMB = 128
TT_VOCAB = 16
B = 1024
S = 200
EPS = 1e-5

NC, NS, L = 2, 16, 16
NW = NC * NS
N = B * S
PER_W = N // NW
G = 80
NBUF = 4
NSTEP = PER_W // G
NJ = EMB // L
CPS = B // G
NPOS = 16


def _rsqrt1(x):
    i = lax.bitcast_convert_type(x, jnp.int32)
    i = 0x5F3759DF - lax.shift_right_arithmetic(i, 1)
    y = lax.bitcast_convert_type(i, jnp.float32)
    return y * (1.5 - (0.5 * x) * y * y)


def _body(ids_hbm, tt_hbm, word_hbm, tt_tab_hbm, pos_hbm, gamma_hbm, beta_hbm,
          out_hbm, ids_v, ttv, pos_v, tt_tab_v, gam_v, bet_v, rowbuf, obuf,
          gsem, osem):
    cid = lax.axis_index("c")
    sid = lax.axis_index("s")
    wid = sid * NC + cid
    base = wid * PER_W
    u0 = wid * NSTEP
    s_lo = pl.multiple_of((u0 // CPS) & ~7, 8)

    pltpu.sync_copy(ids_hbm.at[pl.ds(base, PER_W)], ids_v)
    pltpu.sync_copy(tt_hbm.at[pl.ds(base, PER_W)], ttv.at[pl.ds(0, PER_W)])
    pltpu.sync_copy(pos_hbm.at[pl.ds(0, S)], pos_v)
    pltpu.sync_copy(tt_tab_hbm, tt_tab_v)
    pltpu.sync_copy(gamma_hbm, gam_v)
    pltpu.sync_copy(beta_hbm, bet_v)

    gam = [gam_v[pl.ds(L * j, L)] for j in range(NJ)]
    bet = [bet_v[pl.ds(L * j, L)] for j in range(NJ)]

    def unit_dst(g):
        return out_hbm.at[pl.ds(base + g * G, G)]

    def gather_start(g, slot):
        idx = ids_v.at[pl.ds(g * G, G)]
        pltpu.make_async_copy(word_hbm.at[idx], rowbuf.at[slot],
                              gsem.at[slot]).start()

    def gather_wait(slot):
        pltpu.make_async_copy(
            word_hbm.at[ids_v.at[pl.ds(0, G)]], rowbuf.at[slot],
            gsem.at[slot]).wait()

    def out_start(g, slot):
        pltpu.make_async_copy(obuf.at[slot], unit_dst(g),
                              osem.at[slot]).start()

    def out_wait(g, slot):
        pltpu.make_async_copy(obuf.at[slot], unit_dst(g),
                              osem.at[slot]).wait()

    for b in range(NBUF):
        gather_start(b, b)

    lanes = lax.iota(jnp.int32, L)
    perms = [lanes ^ (1 << k) for k in range(4)]

    dev = None
    for j in range(NJ):
        d = jnp.abs(gam[j] - 1.0) + jnp.abs(bet[j])
        dev = d if dev is None else dev + d
    for k in range(4):
        dev = dev + jnp.take_along_axis(dev, perms[k], axis=0,
                                        mode="promise_in_bounds")
    identity_affine = dev[0] == 0.0

    def step(o, b, affine):
        g = o * NBUF + b

        @plsc.parallel_loop(0, G, unroll=10)
        def token(t):
            tt = ttv[pl.ds(g * G + t, L)][0]
            s = lax.rem(g * G + t, S)
            acc = []
            for j in range(NJ):
                v = (rowbuf[b, t, pl.ds(L * j, L)]
                     + pos_v[s, pl.ds(L * j, L)]) \
                    + tt_tab_v[tt, pl.ds(L * j, L)]
                acc.append(v)
            tot = ((acc[0] + acc[1]) + (acc[2] + acc[3])) \
                + ((acc[4] + acc[5]) + (acc[6] + acc[7]))
            sq = ((acc[0] * acc[0] + acc[1] * acc[1])
                  + (acc[2] * acc[2] + acc[3] * acc[3])) \
                + ((acc[4] * acc[4] + acc[5] * acc[5])
                   + (acc[6] * acc[6] + acc[7] * acc[7]))
            for k in range(4):
                tot = tot + jnp.take_along_axis(
                    tot, perms[k], axis=0, mode="promise_in_bounds")
                sq = sq + jnp.take_along_axis(
                    sq, perms[k], axis=0, mode="promise_in_bounds")
            mean = tot * (1.0 / EMB)
            var = sq * (1.0 / EMB) - mean * mean
            inv = _rsqrt1(var + EPS)
            nb = -mean * inv
            for j in range(NJ):
                r = acc[j] * inv + nb
                if affine:
                    r = r * gam[j] + bet[j]
                obuf[b, t, pl.ds(L * j, L)] = r

    def outer(o, _):
        for b in range(NBUF):
            g = o * NBUF + b
            gather_wait(b)

            @pl.when(o > 0)
            def _():
                out_wait((o - 1) * NBUF + b, b)

            @pl.when(identity_affine)
            def _():
                step(o, b, affine=False)

            @pl.when(jnp.logical_not(identity_affine))
            def _():
                step(o, b, affine=True)

            out_start(g, b)

            @pl.when(o < NSTEP // NBUF - 1)
            def _():
                gather_start(g + NBUF, b)
        return 0

    lax.fori_loop(0, NSTEP // NBUF, outer, 0)

    for b in range(NBUF):
        out_wait(NSTEP - NBUF + b, b)


@jax.jit
def _run(ids, ttids, word_table, tt_tab, pos_tab, gamma, beta):
    k = pl.kernel(
        _body,
        out_type=jax.ShapeDtypeStruct((N, EMB), jnp.float32),
        mesh=plsc.VectorSubcoreMesh(core_axis_name="c", subcore_axis_name="s"),
        compiler_params=pltpu.CompilerParams(needs_layout_passes=False),
        scratch_types=[
            pltpu.VMEM((PER_W,), jnp.int32),
            pltpu.VMEM((PER_W + L,), jnp.int32),
            pltpu.VMEM((S, EMB), jnp.float32),
            pltpu.VMEM((TT_VOCAB, EMB), jnp.float32),
            pltpu.VMEM((EMB,), jnp.float32),
            pltpu.VMEM((EMB,), jnp.float32),
            pltpu.VMEM((NBUF, G, EMB), jnp.float32),
            pltpu.VMEM((NBUF, G, EMB), jnp.float32),
            pltpu.SemaphoreType.DMA((NBUF,)),
            pltpu.SemaphoreType.DMA((NBUF,)),
        ],
    )
    return k(ids, ttids, word_table, tt_tab, pos_tab, gamma, beta)


def kernel(input_ids, token_type_ids, word_table, tt_table, pos_table, gamma,
           beta):
    ids = input_ids.reshape(-1).astype(jnp.int32)
    tts = token_type_ids.reshape(-1).astype(jnp.int32)
    out = _run(ids, tts, word_table, tt_table, pos_table, gamma, beta)
    return out.reshape(B, S, EMB)


# speedup vs baseline: 1.0348x; 1.0348x over previous
import jax
import jax.numpy as jnp
from jax import lax
from jax.experimental import pallas as pl
from jax.experimental.pallas import tpu as pltpu
from jax.experimental.pallas import tpu_sc as plsc

VOCAB = 100000
EMB = 128
TT_VOCAB = 16
B = 1024
S = 200
EPS = 1e-5

NC, NS, L = 2, 16, 16
NW = NC * NS
N = B * S
PER_W = N // NW
G = 80
NBUF = 4
NSTEP = PER_W // G
NJ = EMB // L


def _rsqrt1(x):
    i = lax.bitcast_convert_type(x, jnp.int32)
    i = 0x5F3759DF - lax.shift_right_arithmetic(i, 1)
    y = lax.bitcast_convert_type(i, jnp.float32)
    return y * (1.5 - (0.5 * x) * y * y)


def _body(ids_hbm, tt_hbm, word_hbm, tt_tab_hbm, pos_hbm, gamma_hbm, beta_hbm,
          out_hbm, ids_v, ttv, pos_v, tt_tab_v, gam_v, bet_v, rowbuf, obuf,
          gsem, osem):
    cid = lax.axis_index("c")
    sid = lax.axis_index("s")
    wid = sid * NC + cid
    base = wid * PER_W

    pltpu.sync_copy(ids_hbm.at[pl.ds(base, PER_W)], ids_v)
    pltpu.sync_copy(tt_hbm.at[pl.ds(base, PER_W)], ttv.at[pl.ds(0, PER_W)])
    pltpu.sync_copy(pos_hbm.at[pl.ds(0, S)], pos_v)
    pltpu.sync_copy(tt_tab_hbm, tt_tab_v)
    pltpu.sync_copy(gamma_hbm, gam_v)
    pltpu.sync_copy(beta_hbm, bet_v)

    gam = [gam_v[pl.ds(L * j, L)] for j in range(NJ)]
    bet = [bet_v[pl.ds(L * j, L)] for j in range(NJ)]

    def unit_dst(g):
        return out_hbm.at[pl.ds(base + g * G, G)]

    def gather_start(g, slot):
        idx = ids_v.at[pl.ds(g * G, G)]
        pltpu.make_async_copy(word_hbm.at[idx], rowbuf.at[slot],
                              gsem.at[slot]).start()

    def gather_wait(slot):
        pltpu.make_async_copy(
            word_hbm.at[ids_v.at[pl.ds(0, G)]], rowbuf.at[slot],
            gsem.at[slot]).wait()

    def out_start(g, slot):
        pltpu.make_async_copy(obuf.at[slot], unit_dst(g),
                              osem.at[slot]).start()

    def out_wait(g, slot):
        pltpu.make_async_copy(obuf.at[slot], unit_dst(g),
                              osem.at[slot]).wait()

    for b in range(NBUF):
        gather_start(b, b)

    lanes = lax.iota(jnp.int32, L)
    perms = [lanes ^ (1 << k) for k in range(4)]

    dev = None
    for j in range(NJ):
        d = jnp.abs(gam[j] - 1.0) + jnp.abs(bet[j])
        dev = d if dev is None else dev + d
    for k in range(4):
        dev = dev + jnp.take_along_axis(dev, perms[k], axis=0,
                                        mode="promise_in_bounds")
    identity_affine = dev[0] == 0.0

    def step(o, b, affine):
        g = o * NBUF + b

        @plsc.parallel_loop(0, G, unroll=8)
        def token(t):
            tt = ttv[pl.ds(g * G + t, L)][0]
            s = lax.rem(g * G + t, S)
            acc = []
            for j in range(NJ):
                v = (rowbuf[b, t, pl.ds(L * j, L)]
                     + pos_v[s, pl.ds(L * j, L)]) \
                    + tt_tab_v[tt, pl.ds(L * j, L)]
                acc.append(v)
            tot = ((acc[0] + acc[1]) + (acc[2] + acc[3])) \
                + ((acc[4] + acc[5]) + (acc[6] + acc[7]))
            sq = ((acc[0] * acc[0] + acc[1] * acc[1])
                  + (acc[2] * acc[2] + acc[3] * acc[3])) \
                + ((acc[4] * acc[4] + acc[5] * acc[5])
                   + (acc[6] * acc[6] + acc[7] * acc[7]))
            for k in range(4):
                tot = tot + jnp.take_along_axis(
                    tot, perms[k], axis=0, mode="promise_in_bounds")
                sq = sq + jnp.take_along_axis(
                    sq, perms[k], axis=0, mode="promise_in_bounds")
            mean = tot * (1.0 / EMB)
            var = sq * (1.0 / EMB) - mean * mean
            inv = _rsqrt1(var + EPS)
            nb = -mean * inv
            for j in range(NJ):
                r = acc[j] * inv + nb
                if affine:
                    r = r * gam[j] + bet[j]
                obuf[b, t, pl.ds(L * j, L)] = r

    def outer(o, _):
        for b in range(NBUF):
            g = o * NBUF + b
            gather_wait(b)

            @pl.when(o > 0)
            def _():
                out_wait((o - 1) * NBUF + b, b)

            @pl.when(identity_affine)
            def _():
                step(o, b, affine=False)

            @pl.when(jnp.logical_not(identity_affine))
            def _():
                step(o, b, affine=True)

            out_start(g, b)

            @pl.when(o < NSTEP // NBUF - 1)
            def _():
                gather_start(g + NBUF, b)
        return 0

    lax.fori_loop(0, NSTEP // NBUF, outer, 0)

    for b in range(NBUF):
        out_wait(NSTEP - NBUF + b, b)


@jax.jit
def _run(ids, ttids, word_table, tt_tab, pos_tab, gamma, beta):
    k = pl.kernel(
        _body,
        out_type=jax.ShapeDtypeStruct((N, EMB), jnp.float32),
        mesh=plsc.VectorSubcoreMesh(core_axis_name="c", subcore_axis_name="s"),
        compiler_params=pltpu.CompilerParams(needs_layout_passes=False),
        scratch_types=[
            pltpu.VMEM((PER_W,), jnp.int32),
            pltpu.VMEM((PER_W + L,), jnp.int32),
            pltpu.VMEM((S, EMB), jnp.float32),
            pltpu.VMEM((TT_VOCAB, EMB), jnp.float32),
            pltpu.VMEM((EMB,), jnp.float32),
            pltpu.VMEM((EMB,), jnp.float32),
            pltpu.VMEM((NBUF, G, EMB), jnp.float32),
            pltpu.VMEM((NBUF, G, EMB), jnp.float32),
            pltpu.SemaphoreType.DMA((NBUF,)),
            pltpu.SemaphoreType.DMA((NBUF,)),
        ],
    )
    return k(ids, ttids, word_table, tt_tab, pos_tab, gamma, beta)


def kernel(input_ids, token_type_ids, word_table, tt_table, pos_table, gamma,
           beta):
    ids = input_ids.reshape(-1).astype(jnp.int32)
    tts = token_type_ids.reshape(-1).astype(jnp.int32)
    out = _run(ids, tts, word_table, tt_table, pos_table, gamma, beta)
    return out.reshape(B, S, EMB)
